# Initial kernel scaffold; baseline (speedup 1.0000x reference)
import jax
import jax.numpy as jnp
from jax.experimental import pallas as pl


def kernel(input_ids, attention_mask, position_ids, embed_table):
    raise NotImplementedError("write your pallas kernel here")


# speedup vs baseline: 1.5307x; 1.5307x over previous
import functools

import jax
import jax.numpy as jnp
from jax import lax
from jax.experimental import pallas as pl
from jax.experimental.pallas import tpu as pltpu
from jax.experimental.pallas import tpu_sc as plsc


@functools.lru_cache(maxsize=None)
def _build_gather(n_ids: int, d_model: int):
    info = plsc.get_sparse_core_info()
    n_workers = info.num_cores * info.num_subcores
    b_per_w = n_ids // n_workers
    chunk = 64
    n_chunks = b_per_w // chunk
    mesh = plsc.VectorSubcoreMesh(core_axis_name="c", subcore_axis_name="s")

    @functools.partial(
        pl.kernel,
        mesh=mesh,
        out_type=jax.ShapeDtypeStruct((n_ids, d_model), jnp.float32),
        scratch_types=[
            pltpu.VMEM((b_per_w,), jnp.int32),
            pltpu.VMEM((chunk, d_model), jnp.float32),
            pltpu.SemaphoreType.DMA,
        ],
    )
    def gather_kernel(table_hbm, idx_hbm, out_hbm, idx_v, rows_v, sem):
        wid = lax.axis_index("s") * info.num_cores + lax.axis_index("c")
        base = wid * b_per_w
        pltpu.sync_copy(idx_hbm.at[pl.ds(base, b_per_w)], idx_v)
        for c in range(n_chunks):
            pltpu.async_copy(
                table_hbm.at[idx_v.at[pl.ds(c * chunk, chunk)]], rows_v, sem
            ).wait()
            pltpu.sync_copy(rows_v, out_hbm.at[pl.ds(base + c * chunk, chunk)])

    return gather_kernel


def kernel(input_ids, attention_mask, position_ids, embed_table):
    b, s = input_ids.shape
    _, d = embed_table.shape
    ids_flat = input_ids.reshape(-1).astype(jnp.int32)
    out = _build_gather(b * s, d)(embed_table, ids_flat)
    return out.reshape(b, s, d), attention_mask, position_ids


# speedup vs baseline: 1.6076x; 1.0503x over previous
import functools

import jax
import jax.numpy as jnp
from jax import lax
from jax.experimental import pallas as pl
from jax.experimental.pallas import tpu as pltpu
from jax.experimental.pallas import tpu_sc as plsc


@functools.lru_cache(maxsize=None)
def _build_gather(n_ids: int, d_model: int):
    info = plsc.get_sparse_core_info()
    n_workers = info.num_cores * info.num_subcores
    b_per_w = n_ids // n_workers
    chunk = 32
    nbuf = 2
    n_chunks = b_per_w // chunk
    mesh = plsc.VectorSubcoreMesh(core_axis_name="c", subcore_axis_name="s")

    @functools.partial(
        pl.kernel,
        mesh=mesh,
        out_type=jax.ShapeDtypeStruct((n_ids, d_model), jnp.float32),
        scratch_types=[
            pltpu.VMEM((b_per_w,), jnp.int32),
            *([pltpu.VMEM((chunk, d_model), jnp.float32)] * nbuf),
            *([pltpu.SemaphoreType.DMA] * (2 * nbuf)),
        ],
    )
    def gather_kernel(table_hbm, idx_hbm, out_hbm, idx_v, *scratch):
        rows = scratch[:nbuf]
        gsem = scratch[nbuf : 2 * nbuf]
        osem = scratch[2 * nbuf :]
        wid = lax.axis_index("s") * info.num_cores + lax.axis_index("c")
        base = wid * b_per_w
        pltpu.sync_copy(idx_hbm.at[pl.ds(base, b_per_w)], idx_v)

        def start_gather(c):
            b = c % nbuf
            return pltpu.async_copy(
                table_hbm.at[idx_v.at[pl.ds(c * chunk, chunk)]], rows[b], gsem[b]
            )

        ghandles = {c: start_gather(c) for c in range(nbuf)}
        ohandles = {}
        for c in range(n_chunks):
            b = c % nbuf
            ghandles[c].wait()
            ohandles[c] = pltpu.async_copy(
                rows[b], out_hbm.at[pl.ds(base + c * chunk, chunk)], osem[b]
            )
            nc = c + nbuf
            if nc < n_chunks:
                ohandles[c].wait()
                ghandles[nc] = start_gather(nc)
        for c in range(max(0, n_chunks - nbuf), n_chunks):
            ohandles[c].wait()

    return gather_kernel


def kernel(input_ids, attention_mask, position_ids, embed_table):
    b, s = input_ids.shape
    _, d = embed_table.shape
    ids_flat = input_ids.reshape(-1).astype(jnp.int32)
    out = _build_gather(b * s, d)(embed_table, ids_flat)
    return out.reshape(b, s, d), attention_mask, position_ids
